# baseline (speedup 1.0000x reference)
import jax
import jax.numpy as jnp
from jax.experimental import pallas as pl
from jax.experimental.pallas import tpu as pltpu

A = 21
PSEUDOCOUNT = 0.001
EPS = 1e-9
N_SEQS = 512
L = 256
PB = 64
NB = L // PB
TIA = PB * A
NT = NB * (NB + 1) // 2
INV_LN2 = 1.4426950408889634
LOG2A = 4.392317422778761


def _tri(p):
    i = ((p >= 4).astype(jnp.int32) + (p >= 7).astype(jnp.int32)
         + (p >= 9).astype(jnp.int32))
    j = p - (i * (9 - i)) // 2 + i
    return i, j


def _bin_colsum(t):
    acc = t[:, 0:128]
    for c in range(1, 10):
        acc = acc + t[:, 128 * c:128 * (c + 1)]
    return acc[:, 0:PB] + acc[:, PB:128] + t[:, 20 * PB:TIA]


def _mi_kernel(x_ref, pssm_ref, cons_ref, coev_ref, s1_scr, cnt_scr):
    p = pl.program_id(0)
    i, j = _tri(p)
    xi = x_ref[i]
    xj = x_ref[j]
    counts = jax.lax.dot_general(
        xi, xj, (((1,), (1,)), ((), ())), preferred_element_type=jnp.float32)
    y = counts * (1.0 / N_SEQS) + EPS
    t = y * (jnp.log(y) * INV_LN2)
    part = _bin_colsum(t)
    s = part[0:PB, :]
    for a in range(1, A):
        s = s + part[PB * a:PB * (a + 1), :]
    s1_scr[p] = s

    @pl.when(i == j)
    def _():
        ones = jnp.ones((N_SEQS, 128), jnp.bfloat16)
        cfull = jnp.dot(xi, ones, preferred_element_type=jnp.float32)
        lane = jax.lax.broadcasted_iota(jnp.int32, (PB, 128), 1)
        acc = jnp.where(lane == 0, cfull[0:PB, :], 0.0)
        for a in range(1, A):
            acc = acc + jnp.where(lane == a, cfull[PB * a:PB * (a + 1), :], 0.0)
        cnt_scr[pl.ds(i * PB, PB), :] = acc

    @pl.when(p == NT - 1)
    def _():
        cnt = cnt_scr[...][:, 0:A]
        mean = cnt * (1.0 / N_SEQS)
        freq = mean + PSEUDOCOUNT
        fsum = jnp.sum(freq, axis=1, keepdims=True)
        pssm_ref[...] = jnp.log(freq / fsum * float(A)) * INV_LN2
        fe = mean + EPS
        logp = jnp.log(fe) * INV_LN2
        neg_ent = jnp.sum(fe * logp, axis=1, keepdims=True)
        cons_ref[...] = jnp.transpose(1.0 + neg_ent * (1.0 / LOG2A))
        c_col = jnp.sum((mean + A * EPS) * logp, axis=1, keepdims=True)
        c_row = jnp.transpose(c_col)
        rows = []
        for bi in range(NB):
            tiles = []
            for bj in range(NB):
                lo, hi = min(bi, bj), max(bi, bj)
                tile = s1_scr[lo * (9 - lo) // 2 + hi - lo]
                tiles.append(tile if bi <= bj else jnp.transpose(tile))
            rows.append(jnp.concatenate(tiles, axis=1))
        s1 = jnp.concatenate(rows, axis=0)
        mi = s1 - c_col - c_row
        ii = jax.lax.broadcasted_iota(jnp.int32, (L, L), 0)
        jj = jax.lax.broadcasted_iota(jnp.int32, (L, L), 1)
        mi = jnp.where(ii == jj, 0.0, mi)
        row_mean = jnp.mean(mi, axis=1, keepdims=True)
        col_mean = jnp.mean(mi, axis=0, keepdims=True)
        total = jnp.mean(mi)
        coev_ref[...] = mi - row_mean * col_mean / (total + EPS)


@jax.jit
def kernel(msa):
    mt = msa.astype(jnp.bfloat16).reshape(N_SEQS, L * A).T
    ma = (mt.reshape(NB, PB, A, N_SEQS)
          .transpose(0, 2, 1, 3)
          .reshape(NB, TIA, N_SEQS))
    pssm, cons, coev = pl.pallas_call(
        _mi_kernel,
        grid=(NT,),
        in_specs=[
            pl.BlockSpec((NB, TIA, N_SEQS), lambda p: (0, 0, 0)),
        ],
        out_specs=[
            pl.BlockSpec((L, A), lambda p: (0, 0)),
            pl.BlockSpec((1, L), lambda p: (0, 0)),
            pl.BlockSpec((L, L), lambda p: (0, 0)),
        ],
        out_shape=[
            jax.ShapeDtypeStruct((L, A), jnp.float32),
            jax.ShapeDtypeStruct((1, L), jnp.float32),
            jax.ShapeDtypeStruct((L, L), jnp.float32),
        ],
        scratch_shapes=[
            pltpu.VMEM((NT, PB, PB), jnp.float32),
            pltpu.VMEM((L, 128), jnp.float32),
        ],
        compiler_params=pltpu.CompilerParams(
            dimension_semantics=("arbitrary",),
            vmem_limit_bytes=56 * 1024 * 1024,
        ),
        name="mi_tiles",
    )(ma)
    return pssm, cons.reshape(L), coev


# speedup vs baseline: 1.4398x; 1.4398x over previous
import jax
import jax.numpy as jnp
from jax.experimental import pallas as pl
from jax.experimental.pallas import tpu as pltpu

A = 21
PSEUDOCOUNT = 0.001
EPS = 1e-9
N_SEQS = 512
L = 256
PB = 64
NB = L // PB
TIA = PB * A
NT = NB * (NB + 1) // 2
INV_LN2 = 1.4426950408889634
LOG2A = 4.392317422778761


def _tri(p):
    i = ((p >= 4).astype(jnp.int32) + (p >= 7).astype(jnp.int32)
         + (p >= 9).astype(jnp.int32))
    j = p - (i * (9 - i)) // 2 + i
    return i, j


def _bin_colsum(t):
    acc = t[:, 0:128]
    for c in range(1, 10):
        acc = acc + t[:, 128 * c:128 * (c + 1)]
    return acc[:, 0:PB] + acc[:, PB:128] + t[:, 20 * PB:TIA]


def _mi_kernel(x_ref, pssm_ref, cons_ref, coev_ref, s1_scr, cnt_scr):
    p = pl.program_id(0)
    i, j = _tri(p)
    xi = x_ref[i]
    xj = x_ref[j]
    counts = jax.lax.dot_general(
        xi, xj, (((1,), (1,)), ((), ())), preferred_element_type=jnp.float32)
    y = counts * (1.0 / N_SEQS) + EPS
    t = y * (jnp.log(y) * INV_LN2)
    part = _bin_colsum(t)
    s = part[0:PB, :]
    for a in range(1, A):
        s = s + part[PB * a:PB * (a + 1), :]
    s1_scr[p] = s

    @pl.when(i == j)
    def _():
        ones = jnp.ones((N_SEQS, 128), jnp.bfloat16)
        cfull = jnp.dot(xi, ones, preferred_element_type=jnp.float32)
        lane = jax.lax.broadcasted_iota(jnp.int32, (PB, 128), 1)
        acc = jnp.where(lane == 0, cfull[0:PB, :], 0.0)
        for a in range(1, A):
            acc = acc + jnp.where(lane == a, cfull[PB * a:PB * (a + 1), :], 0.0)
        cnt_scr[pl.ds(i * PB, PB), :] = acc

    @pl.when(p == NT - 1)
    def _():
        cnt = cnt_scr[...][:, 0:A]
        mean = cnt * (1.0 / N_SEQS)
        freq = mean + PSEUDOCOUNT
        fsum = jnp.sum(freq, axis=1, keepdims=True)
        pssm_ref[...] = jnp.log(freq / fsum * float(A)) * INV_LN2
        fe = mean + EPS
        logp = jnp.log(fe) * INV_LN2
        neg_ent = jnp.sum(fe * logp, axis=1, keepdims=True)
        cons_ref[...] = jnp.transpose(1.0 + neg_ent * (1.0 / LOG2A))
        c_col = jnp.sum((mean + A * EPS) * logp, axis=1, keepdims=True)
        c_row = jnp.transpose(c_col)
        rows = []
        for bi in range(NB):
            tiles = []
            for bj in range(NB):
                lo, hi = min(bi, bj), max(bi, bj)
                tile = s1_scr[lo * (9 - lo) // 2 + hi - lo]
                tiles.append(tile if bi <= bj else jnp.transpose(tile))
            rows.append(jnp.concatenate(tiles, axis=1))
        s1 = jnp.concatenate(rows, axis=0)
        mi = s1 - c_col - c_row
        ii = jax.lax.broadcasted_iota(jnp.int32, (L, L), 0)
        jj = jax.lax.broadcasted_iota(jnp.int32, (L, L), 1)
        mi = jnp.where(ii == jj, 0.0, mi)
        row_mean = jnp.mean(mi, axis=1, keepdims=True)
        col_mean = jnp.mean(mi, axis=0, keepdims=True)
        total = jnp.mean(mi)
        coev_ref[...] = mi - row_mean * col_mean / (total + EPS)


@jax.jit
def kernel(msa):
    ma = (msa.astype(jnp.bfloat16)
          .reshape(N_SEQS, NB, PB, A)
          .transpose(1, 3, 2, 0)
          .reshape(NB, TIA, N_SEQS))
    pssm, cons, coev = pl.pallas_call(
        _mi_kernel,
        grid=(NT,),
        in_specs=[
            pl.BlockSpec((NB, TIA, N_SEQS), lambda p: (0, 0, 0)),
        ],
        out_specs=[
            pl.BlockSpec((L, A), lambda p: (0, 0)),
            pl.BlockSpec((1, L), lambda p: (0, 0)),
            pl.BlockSpec((L, L), lambda p: (0, 0)),
        ],
        out_shape=[
            jax.ShapeDtypeStruct((L, A), jnp.float32),
            jax.ShapeDtypeStruct((1, L), jnp.float32),
            jax.ShapeDtypeStruct((L, L), jnp.float32),
        ],
        scratch_shapes=[
            pltpu.VMEM((NT, PB, PB), jnp.float32),
            pltpu.VMEM((L, 128), jnp.float32),
        ],
        compiler_params=pltpu.CompilerParams(
            dimension_semantics=("arbitrary",),
            vmem_limit_bytes=56 * 1024 * 1024,
        ),
        name="mi_tiles",
    )(ma)
    return pssm, cons.reshape(L), coev
